# baseline (speedup 1.0000x reference)
import jax
import jax.numpy as jnp
from jax.experimental import pallas as pl
from jax.experimental.pallas import tpu as pltpu


_ROWS_PER_BLOCK = 256


def _onehot_argmax_block(x_ref, o_ref):
    x = x_ref[...]
    idx = jnp.argmax(x, axis=-1).astype(jnp.int32)
    iota = jax.lax.broadcasted_iota(jnp.int32, x.shape, 1)
    o_ref[...] = (iota == idx[:, None]).astype(x.dtype)


def kernel(inputs):
    b, t, m = inputs.shape
    x2 = inputs.reshape(b * t, m)
    n = b * t
    grid = (n // _ROWS_PER_BLOCK,)
    out = pl.pallas_call(
        _onehot_argmax_block,
        grid=grid,
        in_specs=[pl.BlockSpec((_ROWS_PER_BLOCK, m), lambda i: (i, 0))],
        out_specs=pl.BlockSpec((_ROWS_PER_BLOCK, m), lambda i: (i, 0)),
        out_shape=jax.ShapeDtypeStruct((n, m), inputs.dtype),
        compiler_params=pltpu.CompilerParams(
            dimension_semantics=("parallel",),
        ),
    )(x2)
    return out.reshape(b, t, m)


# speedup vs baseline: 1.6833x; 1.6833x over previous
import jax
import jax.numpy as jnp
from jax.experimental import pallas as pl
from jax.experimental.pallas import tpu as pltpu


_ROWS_PER_BLOCK = 1024


def _onehot_argmax_block(x_ref, o_ref):
    x = x_ref[...]
    idx = jnp.argmax(x, axis=-1).astype(jnp.int32)
    iota = jax.lax.broadcasted_iota(jnp.int32, x.shape, 1)
    o_ref[...] = (iota == idx[:, None]).astype(x.dtype)


def kernel(inputs):
    b, t, m = inputs.shape
    x2 = inputs.reshape(b * t, m)
    n = b * t
    grid = (n // _ROWS_PER_BLOCK,)
    out = pl.pallas_call(
        _onehot_argmax_block,
        grid=grid,
        in_specs=[pl.BlockSpec((_ROWS_PER_BLOCK, m), lambda i: (i, 0))],
        out_specs=pl.BlockSpec((_ROWS_PER_BLOCK, m), lambda i: (i, 0)),
        out_shape=jax.ShapeDtypeStruct((n, m), inputs.dtype),
        compiler_params=pltpu.CompilerParams(
            dimension_semantics=("parallel",),
        ),
    )(x2)
    return out.reshape(b, t, m)


# speedup vs baseline: 1.7613x; 1.0463x over previous
import jax
import jax.numpy as jnp
from jax.experimental import pallas as pl
from jax.experimental.pallas import tpu as pltpu


_ROWS_PER_BLOCK = 2048


def _onehot_argmax_block(x_ref, o_ref):
    x = x_ref[...]
    idx = jnp.argmax(x, axis=-1).astype(jnp.int32)
    iota = jax.lax.broadcasted_iota(jnp.int32, x.shape, 1)
    o_ref[...] = (iota == idx[:, None]).astype(x.dtype)


def kernel(inputs):
    b, t, m = inputs.shape
    x2 = inputs.reshape(b * t, m)
    n = b * t
    grid = (n // _ROWS_PER_BLOCK,)
    out = pl.pallas_call(
        _onehot_argmax_block,
        grid=grid,
        in_specs=[pl.BlockSpec((_ROWS_PER_BLOCK, m), lambda i: (i, 0))],
        out_specs=pl.BlockSpec((_ROWS_PER_BLOCK, m), lambda i: (i, 0)),
        out_shape=jax.ShapeDtypeStruct((n, m), inputs.dtype),
        compiler_params=pltpu.CompilerParams(
            dimension_semantics=("parallel",),
        ),
    )(x2)
    return out.reshape(b, t, m)


# speedup vs baseline: 1.8131x; 1.0294x over previous
import jax
import jax.numpy as jnp
from jax.experimental import pallas as pl
from jax.experimental.pallas import tpu as pltpu


_ROWS_PER_BLOCK = 3072


def _onehot_argmax_block(x_ref, o_ref):
    x = x_ref[...]
    idx = jnp.argmax(x, axis=-1).astype(jnp.int32)
    iota = jax.lax.broadcasted_iota(jnp.int32, x.shape, 1)
    o_ref[...] = (iota == idx[:, None]).astype(x.dtype)


def kernel(inputs):
    b, t, m = inputs.shape
    x2 = inputs.reshape(b * t, m)
    n = b * t
    grid = (n // _ROWS_PER_BLOCK,)
    out = pl.pallas_call(
        _onehot_argmax_block,
        grid=grid,
        in_specs=[pl.BlockSpec((_ROWS_PER_BLOCK, m), lambda i: (i, 0))],
        out_specs=pl.BlockSpec((_ROWS_PER_BLOCK, m), lambda i: (i, 0)),
        out_shape=jax.ShapeDtypeStruct((n, m), inputs.dtype),
        compiler_params=pltpu.CompilerParams(
            dimension_semantics=("parallel",),
        ),
    )(x2)
    return out.reshape(b, t, m)
